# Initial kernel scaffold; baseline (speedup 1.0000x reference)
import jax
import jax.numpy as jnp
from jax.experimental import pallas as pl


def kernel(x, fc_weight, bias):
    raise NotImplementedError("write your pallas kernel here")


# speedup vs baseline: 1.2098x; 1.2098x over previous
import functools

import jax
import jax.numpy as jnp
from jax import lax
from jax.experimental import pallas as pl
from jax.experimental.pallas import tpu as pltpu
from jax.experimental.pallas import tpu_sc as plsc

BATCH = 4096
FIELDS = 26
NC = 2
NS = 16
NW = NC * NS
BPW = BATCH // NW
LANES = 16

_mesh = plsc.VectorSubcoreMesh(core_axis_name="c", subcore_axis_name="s")


@functools.partial(
    pl.kernel,
    mesh=_mesh,
    out_type=jax.ShapeDtypeStruct((BATCH,), jnp.float32),
    scratch_types=[
        pltpu.VMEM((FIELDS, BPW), jnp.int32),
        pltpu.VMEM((FIELDS, BPW), jnp.float32),
        pltpu.VMEM((BPW,), jnp.float32),
        pltpu.VMEM((LANES,), jnp.float32),
        pltpu.SemaphoreType.DMA,
    ],
)
def _sc_kernel(table_hbm, xr_hbm, bias_hbm, out_hbm,
               idx_v, vals_v, out_v, bias_v, sem):
    wid = lax.axis_index("s") * NC + lax.axis_index("c")
    base = wid * BPW

    pltpu.sync_copy(xr_hbm.at[wid], idx_v)
    pltpu.sync_copy(bias_hbm, bias_v)

    handles = []
    for j in range(FIELDS):
        handles.append(
            pltpu.async_copy(table_hbm.at[idx_v.at[j]], vals_v.at[j], sem))
    for h in handles:
        h.wait()

    bias_vec = bias_v[...]
    for k in range(BPW // LANES):
        acc = bias_vec
        for j in range(FIELDS):
            acc = acc + vals_v[j, pl.ds(k * LANES, LANES)]
        out_v[pl.ds(k * LANES, LANES)] = acc

    pltpu.sync_copy(out_v, out_hbm.at[pl.ds(base, BPW)])


def kernel(x, fc_weight, bias):
    table = fc_weight.reshape(-1)
    xr = x.T.reshape(FIELDS, NW, BPW).transpose(1, 0, 2)
    bias16 = jnp.broadcast_to(bias, (LANES,)).astype(jnp.float32)
    out = _sc_kernel(table, xr, bias16)
    return out.reshape(BATCH, 1)


# speedup vs baseline: 1.2903x; 1.0666x over previous
import functools

import jax
import jax.numpy as jnp
from jax import lax
from jax.experimental import pallas as pl
from jax.experimental.pallas import tpu as pltpu
from jax.experimental.pallas import tpu_sc as plsc

BATCH = 4096
FIELDS = 26
NC = 2
NS = 16
NW = NC * NS
BPW = BATCH // NW
LANES = 16

_mesh = plsc.VectorSubcoreMesh(core_axis_name="c", subcore_axis_name="s")


@functools.partial(
    pl.kernel,
    mesh=_mesh,
    out_type=jax.ShapeDtypeStruct((BATCH,), jnp.float32),
    scratch_types=[
        pltpu.VMEM((FIELDS, BPW), jnp.int32),
        pltpu.VMEM((FIELDS, BPW), jnp.float32),
        pltpu.VMEM((BPW,), jnp.float32),
        pltpu.VMEM((LANES,), jnp.float32),
        pltpu.SemaphoreType.DMA,
    ],
)
def _sc_kernel(table_hbm, xr_hbm, bias_hbm, out_hbm,
               idxt_v, vals_v, out_v, bias_s, sem):
    wid = lax.axis_index("s") * NC + lax.axis_index("c")
    base = wid * BPW

    pltpu.sync_copy(bias_hbm, bias_s.at[pl.ds(0, 1)])
    pltpu.sync_copy(xr_hbm.at[wid], idxt_v)

    handles = []
    for j in range(FIELDS):
        handles.append(
            pltpu.async_copy(table_hbm.at[idxt_v.at[j]], vals_v.at[j], sem))

    bvec = lax.broadcast(bias_s[...][0], (LANES,))
    accs = [bvec for _ in range(BPW // LANES)]
    for j in range(FIELDS):
        handles[j].wait()
        for k in range(BPW // LANES):
            accs[k] = accs[k] + vals_v[j, pl.ds(k * LANES, LANES)]
    for k in range(BPW // LANES):
        out_v[pl.ds(k * LANES, LANES)] = accs[k]

    pltpu.sync_copy(out_v, out_hbm.at[pl.ds(base, BPW)])


def kernel(x, fc_weight, bias):
    table = fc_weight.reshape(-1)
    xr = x.reshape(NW, BPW, FIELDS).swapaxes(1, 2)
    out = _sc_kernel(table, xr, bias)
    return out.reshape(BATCH, 1)


# speedup vs baseline: 1.3430x; 1.0408x over previous
import functools

import jax
import jax.numpy as jnp
from jax import lax
from jax.experimental import pallas as pl
from jax.experimental.pallas import tpu as pltpu
from jax.experimental.pallas import tpu_sc as plsc

BATCH = 4096
FIELDS = 26
NC = 2
NS = 16
NW = NC * NS
BPW = BATCH // NW
LANES = 16

_mesh = plsc.VectorSubcoreMesh(core_axis_name="c", subcore_axis_name="s")


@functools.partial(
    pl.kernel,
    mesh=_mesh,
    out_type=jax.ShapeDtypeStruct((BATCH,), jnp.float32),
    scratch_types=[
        pltpu.VMEM((FIELDS, BPW), jnp.int32),
        pltpu.VMEM((FIELDS, BPW), jnp.float32),
        pltpu.VMEM((BPW,), jnp.float32),
        pltpu.VMEM((LANES,), jnp.float32),
        pltpu.SemaphoreType.DMA,
    ],
)
def _sc_kernel(table_hbm, xr_hbm, bias_hbm, out_hbm,
               idxt_v, vals_v, out_v, bias_s, sem):
    wid = lax.axis_index("s") * NC + lax.axis_index("c")
    base = wid * BPW

    pltpu.sync_copy(bias_hbm, bias_s.at[pl.ds(0, 1)])
    pltpu.sync_copy(xr_hbm.at[:, pl.ds(base, BPW)], idxt_v)

    handles = []
    for j in range(FIELDS):
        handles.append(
            pltpu.async_copy(table_hbm.at[0].at[idxt_v.at[j]], vals_v.at[j],
                             sem))

    bvec = lax.broadcast(bias_s[...][0], (LANES,))
    accs = [bvec for _ in range(BPW // LANES)]
    for j in range(FIELDS):
        handles[j].wait()
        for k in range(BPW // LANES):
            accs[k] = accs[k] + vals_v[j, pl.ds(k * LANES, LANES)]
    for k in range(BPW // LANES):
        out_v[pl.ds(k * LANES, LANES)] = accs[k]

    pltpu.sync_copy(out_v, out_hbm.at[pl.ds(base, BPW)])


def kernel(x, fc_weight, bias):
    table = fc_weight.T
    xr = x.T
    out = _sc_kernel(table, xr, bias)
    return out.reshape(BATCH, 1)
